# baseline (speedup 1.0000x reference)
import functools

import jax
import jax.numpy as jnp
from jax import lax
from jax.experimental import pallas as pl
from jax.experimental.pallas import tpu as pltpu
from jax.experimental.pallas import tpu_sc as plsc

_EMBED = 64
_LANES = 16
_NC = 2
_NS = 16
_NW = _NC * _NS
_CHUNK = 128
_NBUF = 8
_LEAD = 6


@functools.lru_cache(maxsize=None)
def _make_emb_kernel(ntok: int):
    npw = ntok // _NW
    nchunk = npw // _CHUNK
    assert nchunk % _NBUF == 0 and nchunk // _NBUF >= 3
    mesh = plsc.VectorSubcoreMesh(core_axis_name="c", subcore_axis_name="s")

    @functools.partial(
        pl.kernel,
        out_type=jax.ShapeDtypeStruct((ntok, _EMBED), jnp.float32),
        mesh=mesh,
        scratch_types=[
            pltpu.VMEM((nchunk, _CHUNK), jnp.int32),
            pltpu.VMEM((_NBUF, _CHUNK, _EMBED), jnp.float32),
            pltpu.SemaphoreType.DMA((_NBUF,)),
            pltpu.SemaphoreType.DMA((_NBUF,)),
        ],
        compiler_params=pltpu.CompilerParams(use_tc_tiling_on_sc=False),
    )
    def emb(idx_hbm, table_hbm, out_hbm, idx_v, rows_v, gsem, ssem):
        wid = lax.axis_index("s") * _NC + lax.axis_index("c")
        base = wid * npw
        pltpu.sync_copy(idx_hbm.at[wid], idx_v)

        def gather_issue(k, b):
            pltpu.async_copy(table_hbm.at[idx_v.at[k]], rows_v.at[b], gsem.at[b])

        def gather_wait(b):
            pltpu.make_async_copy(
                table_hbm.at[pl.ds(0, _CHUNK)], rows_v.at[b], gsem.at[b]
            ).wait()

        def scatter_issue(k, b):
            pltpu.async_copy(
                rows_v.at[b], out_hbm.at[pl.ds(base + k * _CHUNK, _CHUNK)], ssem.at[b]
            )

        def scatter_wait(b):
            pltpu.make_async_copy(
                rows_v.at[b], out_hbm.at[pl.ds(base, _CHUNK)], ssem.at[b]
            ).wait()

        def scale(b):
            @pl.loop(0, _CHUNK, unroll=8)
            def _(i):
                for j in range(_EMBED // _LANES):
                    sl = pl.ds(j * _LANES, _LANES)
                    rows_v[b, i, sl] = rows_v[b, i, sl] * 8.0

        for g in range(_LEAD):
            gather_issue(g, g)

        for g in range(_NBUF):
            b = g
            gather_wait(b)
            scale(b)
            scatter_issue(g, b)
            if g >= 2:
                scatter_wait((g - 2) % _NBUF)
            gather_issue(g + _LEAD, (g + _LEAD) % _NBUF)

        @pl.loop(1, nchunk // _NBUF - 1)
        def _(s):
            k0 = s * _NBUF
            for b in range(_NBUF):
                k = k0 + b
                gather_wait(b)
                scale(b)
                scatter_issue(k, b)
                scatter_wait((b + _LEAD) % _NBUF)
                gather_issue(k + _LEAD, (b + _LEAD) % _NBUF)

        for g in range(nchunk - _NBUF, nchunk):
            b = g % _NBUF
            gather_wait(b)
            scale(b)
            scatter_issue(g, b)
            if g + _LEAD < nchunk:
                scatter_wait((b + _LEAD) % _NBUF)
                gather_issue(g + _LEAD, (b + _LEAD) % _NBUF)

        for b in range(_NBUF):
            scatter_wait(b)

    return emb


def kernel(x, input_embedding_table):
    b, l = x.shape
    ntok = b * l
    idx = x.reshape(_NW, ntok // _NW // _CHUNK, _CHUNK).astype(jnp.int32)
    out = _make_emb_kernel(ntok)(idx, input_embedding_table)
    return out.reshape(b, l, _EMBED)


# speedup vs baseline: 1.0296x; 1.0296x over previous
import functools

import jax
import jax.numpy as jnp
from jax import lax
from jax.experimental import pallas as pl
from jax.experimental.pallas import tpu as pltpu
from jax.experimental.pallas import tpu_sc as plsc

_EMBED = 64
_LANES = 16
_NC = 2
_NS = 16
_NW = _NC * _NS
_CHUNK = 128
_NBUF = 8
_LEAD = 6


@functools.lru_cache(maxsize=None)
def _make_emb_kernel(nl: int, nb: int):
    nchunk = nl
    assert nb == _NW * _CHUNK
    assert nchunk % _NBUF == 0 and nchunk // _NBUF >= 3
    mesh = plsc.VectorSubcoreMesh(core_axis_name="c", subcore_axis_name="s")

    @functools.partial(
        pl.kernel,
        out_type=jax.ShapeDtypeStruct((nl * nb, _EMBED), jnp.float32),
        mesh=mesh,
        scratch_types=[
            pltpu.VMEM((nchunk, _CHUNK), jnp.int32),
            pltpu.VMEM((_NBUF, _CHUNK, _EMBED), jnp.float32),
            pltpu.SemaphoreType.DMA((_NBUF,)),
            pltpu.SemaphoreType.DMA((_NBUF,)),
        ],
        compiler_params=pltpu.CompilerParams(use_tc_tiling_on_sc=False),
    )
    def emb(idx_hbm, table_hbm, out_hbm, idx_v, rows_v, gsem, ssem):
        wid = lax.axis_index("s") * _NC + lax.axis_index("c")
        base = wid * _CHUNK
        pltpu.sync_copy(idx_hbm.at[:, pl.ds(base, _CHUNK)], idx_v)

        def gather_issue(k, b):
            pltpu.async_copy(table_hbm.at[idx_v.at[k]], rows_v.at[b], gsem.at[b])

        def gather_wait(b):
            pltpu.make_async_copy(
                table_hbm.at[pl.ds(0, _CHUNK)], rows_v.at[b], gsem.at[b]
            ).wait()

        def scatter_issue(k, b):
            pltpu.async_copy(
                rows_v.at[b], out_hbm.at[pl.ds(k * nb + base, _CHUNK)], ssem.at[b]
            )

        def scatter_wait(b):
            pltpu.make_async_copy(
                rows_v.at[b], out_hbm.at[pl.ds(base, _CHUNK)], ssem.at[b]
            ).wait()

        def scale(b):
            @pl.loop(0, _CHUNK, unroll=8)
            def _(i):
                for j in range(_EMBED // _LANES):
                    sl = pl.ds(j * _LANES, _LANES)
                    rows_v[b, i, sl] = rows_v[b, i, sl] * 8.0

        for g in range(_LEAD):
            gather_issue(g, g)

        for g in range(_NBUF):
            b = g
            gather_wait(b)
            scale(b)
            scatter_issue(g, b)
            if g >= 2:
                scatter_wait((g - 2) % _NBUF)
            gather_issue(g + _LEAD, (g + _LEAD) % _NBUF)

        @pl.loop(1, nchunk // _NBUF - 1)
        def _(s):
            k0 = s * _NBUF
            for b in range(_NBUF):
                k = k0 + b
                gather_wait(b)
                scale(b)
                scatter_issue(k, b)
                scatter_wait((b + _LEAD) % _NBUF)
                gather_issue(k + _LEAD, (b + _LEAD) % _NBUF)

        for g in range(nchunk - _NBUF, nchunk):
            b = g % _NBUF
            gather_wait(b)
            scale(b)
            scatter_issue(g, b)
            if g + _LEAD < nchunk:
                scatter_wait((b + _LEAD) % _NBUF)
                gather_issue(g + _LEAD, (b + _LEAD) % _NBUF)

        for b in range(_NBUF):
            scatter_wait(b)

    return emb


def kernel(x, input_embedding_table):
    nb, nl = x.shape
    xt = x.T
    out = _make_emb_kernel(nl, nb)(xt, input_embedding_table)
    return out.reshape(nl, nb, _EMBED).transpose(1, 0, 2)


# speedup vs baseline: 1.1319x; 1.0995x over previous
import functools

import jax
import jax.numpy as jnp
from jax import lax
from jax.experimental import pallas as pl
from jax.experimental.pallas import tpu as pltpu
from jax.experimental.pallas import tpu_sc as plsc

_EMBED = 64
_LANES = 16
_NC = 2
_NS = 16
_NW = _NC * _NS
_CHUNK = 128
_NBUF = 8
_LEAD = 6


@functools.lru_cache(maxsize=None)
def _make_emb_kernel(nl: int, nb: int):
    nlt = nl // _NBUF
    assert nb == _NW * _CHUNK and nl % _NBUF == 0 and nlt >= 3
    mesh = plsc.VectorSubcoreMesh(core_axis_name="c", subcore_axis_name="s")

    @functools.partial(
        pl.kernel,
        out_type=jax.ShapeDtypeStruct((nb, nl * _EMBED), jnp.float32),
        mesh=mesh,
        scratch_types=[
            pltpu.VMEM((nlt, 1, _NBUF, _CHUNK), jnp.int32),
            pltpu.VMEM((_NBUF, _CHUNK, _EMBED), jnp.float32),
            pltpu.SemaphoreType.DMA((_NBUF,)),
            pltpu.SemaphoreType.DMA((_NBUF,)),
        ],
        compiler_params=pltpu.CompilerParams(use_tc_tiling_on_sc=False),
    )
    def emb(idx_hbm, table_hbm, out_hbm, idx_v, rows_v, gsem, ssem):
        wid = lax.axis_index("s") * _NC + lax.axis_index("c")
        col = wid * _CHUNK
        pltpu.sync_copy(idx_hbm.at[:, pl.ds(wid, 1)], idx_v)

        def gather_issue(lt, ls, b):
            pltpu.async_copy(
                table_hbm.at[idx_v.at[lt, 0, ls]], rows_v.at[b], gsem.at[b]
            )

        def gather_wait(b):
            pltpu.make_async_copy(
                table_hbm.at[pl.ds(0, _CHUNK)], rows_v.at[b], gsem.at[b]
            ).wait()

        def scatter_issue(lt, ls, b):
            l = lt * _NBUF + ls
            pltpu.async_copy(
                rows_v.at[b],
                out_hbm.at[pl.ds(col, _CHUNK), pl.ds(l * _EMBED, _EMBED)],
                ssem.at[b],
            )

        def scatter_wait(b):
            pltpu.make_async_copy(
                rows_v.at[b],
                out_hbm.at[pl.ds(0, _CHUNK), pl.ds(0, _EMBED)],
                ssem.at[b],
            ).wait()

        def scale(b):
            @pl.loop(0, _CHUNK, unroll=8)
            def _(i):
                for j in range(_EMBED // _LANES):
                    sl = pl.ds(j * _LANES, _LANES)
                    rows_v[b, i, sl] = rows_v[b, i, sl] * 8.0

        for ls in range(_LEAD):
            gather_issue(0, ls, ls)

        for ls in range(_NBUF):
            gather_wait(ls)
            scale(ls)
            scatter_issue(0, ls, ls)
            if ls >= 2:
                scatter_wait((ls - 2) % _NBUF)
            gather_issue((ls + _LEAD) // _NBUF, (ls + _LEAD) % _NBUF,
                         (ls + _LEAD) % _NBUF)

        @pl.loop(1, nlt - 1)
        def _(lt):
            for ls in range(_NBUF):
                gather_wait(ls)
                scale(ls)
                scatter_issue(lt, ls, ls)
                scatter_wait((ls + _LEAD) % _NBUF)
                gather_issue(lt + (ls + _LEAD) // _NBUF, (ls + _LEAD) % _NBUF,
                             (ls + _LEAD) % _NBUF)

        for ls in range(_NBUF):
            gather_wait(ls)
            scale(ls)
            scatter_issue(nlt - 1, ls, ls)
            if ls + _LEAD < _NBUF:
                scatter_wait(ls + _LEAD)
                gather_issue(nlt - 1, ls + _LEAD, ls + _LEAD)

        for b in range(_NBUF):
            scatter_wait(b)

    return emb


def kernel(x, input_embedding_table):
    nb, nl = x.shape
    idx4 = (
        x.T.reshape(nl // _NBUF, _NBUF, _NW, _CHUNK).transpose(0, 2, 1, 3)
    )
    out = _make_emb_kernel(nl, nb)(idx4, input_embedding_table)
    return out.reshape(nb, nl, _EMBED)
